# baseline (speedup 1.0000x reference)
import functools

import jax
import jax.numpy as jnp
from jax import lax
from jax.experimental import pallas as pl
from jax.experimental.pallas import tpu as pltpu
from jax.experimental.pallas import tpu_sc as plsc


def _make_sc_lookup(n_tokens: int, d: int):
    info = plsc.get_sparse_core_info()
    nw = info.num_cores * info.num_subcores
    assert n_tokens % (8 * nw) == 0
    n_per_w = n_tokens // nw
    chunk = 64
    while n_per_w % chunk:
        chunk //= 2
    n_chunks = n_per_w // chunk
    mesh = plsc.VectorSubcoreMesh(core_axis_name="c", subcore_axis_name="s")

    @functools.partial(
        pl.kernel,
        mesh=mesh,
        out_type=jax.ShapeDtypeStruct((n_tokens, d), jnp.float32),
        scratch_types=[
            pltpu.VMEM((n_per_w,), jnp.int32),
            pltpu.VMEM((chunk, d), jnp.float32),
            pltpu.SemaphoreType.DMA,
        ],
    )
    def lookup(table_hbm, idx_hbm, out_hbm, idx_v, rows_v, gsem):
        wid = lax.axis_index("s") * info.num_cores + lax.axis_index("c")
        base = wid * n_per_w
        pltpu.sync_copy(idx_hbm.at[pl.ds(base, n_per_w)], idx_v)

        def body(ci, _):
            off = ci * chunk
            pltpu.async_copy(
                table_hbm.at[idx_v.at[pl.ds(off, chunk)]], rows_v, gsem
            ).wait()
            pltpu.sync_copy(rows_v, out_hbm.at[pl.ds(base + off, chunk)])
            return 0

        lax.fori_loop(0, n_chunks, body, 0)

    return lookup


def _make_tc_lookup(n_tokens: int, vocab_pad: int, d: int, block: int):
    n_blocks = n_tokens // block

    def body(tok_ref, hi_ref, lo_ref, out_ref):
        tok = tok_ref[0, 0, :]
        oh = (
            tok[:, None]
            == lax.broadcasted_iota(jnp.int32, (block, vocab_pad), 1)
        ).astype(jnp.bfloat16)
        acc = jnp.dot(oh, hi_ref[...], preferred_element_type=jnp.float32)
        acc += jnp.dot(oh, lo_ref[...], preferred_element_type=jnp.float32)
        out_ref[...] = acc

    return pl.pallas_call(
        body,
        grid=(n_blocks,),
        in_specs=[
            pl.BlockSpec((1, 1, block), lambda i: (i, 0, 0)),
            pl.BlockSpec((vocab_pad, d), lambda i: (0, 0)),
            pl.BlockSpec((vocab_pad, d), lambda i: (0, 0)),
        ],
        out_specs=pl.BlockSpec((block, d), lambda i: (i, 0)),
        out_shape=jax.ShapeDtypeStruct((n_tokens, d), jnp.float32),
    )


def kernel(tokens, attention_mask, table):
    b, s = tokens.shape
    vocab, d = table.shape
    n = b * s
    idx = tokens.reshape(n).astype(jnp.int32)

    block = 512
    n_tc = 14336
    n_sc = n - n_tc

    vocab_pad = -(-vocab // 256) * 256
    table_pad = jnp.pad(table, ((0, vocab_pad - vocab), (0, 0)))
    hi = table_pad.astype(jnp.bfloat16)
    lo = (table_pad - hi.astype(jnp.float32)).astype(jnp.bfloat16)

    out_sc = _make_sc_lookup(n_sc, d)(table, idx[:n_sc])
    out_tc = _make_tc_lookup(n_tc, vocab_pad, d, block)(
        idx[n_sc:].reshape(n_tc // block, 1, block), hi, lo
    )
    out = jnp.concatenate([out_sc, out_tc], axis=0)
    return out.reshape(b, s, d)


# speedup vs baseline: 1.5824x; 1.5824x over previous
import functools

import jax
import jax.numpy as jnp
from jax import lax
from jax.experimental import pallas as pl
from jax.experimental.pallas import tpu as pltpu
from jax.experimental.pallas import tpu_sc as plsc


def _make_lookup(n_tokens: int, d: int):
    info = plsc.get_sparse_core_info()
    nw = info.num_cores * info.num_subcores
    assert n_tokens % (8 * nw) == 0
    n_per_w = n_tokens // nw
    chunk = 32
    while n_per_w % (4 * chunk):
        chunk //= 2
    n_chunks = n_per_w // chunk
    mesh = plsc.VectorSubcoreMesh(core_axis_name="c", subcore_axis_name="s")

    @functools.partial(
        pl.kernel,
        mesh=mesh,
        out_type=jax.ShapeDtypeStruct((n_tokens, d), jnp.float32),
        scratch_types=[
            pltpu.VMEM((n_per_w,), jnp.int32),
            pltpu.VMEM((chunk, d), jnp.float32),
            pltpu.VMEM((chunk, d), jnp.float32),
            pltpu.VMEM((chunk, d), jnp.float32),
            pltpu.VMEM((chunk, d), jnp.float32),
            pltpu.SemaphoreType.DMA,
            pltpu.SemaphoreType.DMA,
        ],
    )
    def lookup(table_hbm, idx_hbm, out_hbm, idx_v, b0, b1, b2, b3, gsem, wsem):
        wid = lax.axis_index("s") * info.num_cores + lax.axis_index("c")
        base = wid * n_per_w
        pltpu.sync_copy(idx_hbm.at[pl.ds(base, n_per_w)], idx_v)

        bufs = (b0, b1, b2, b3)

        def start_gather(ci, b):
            pltpu.async_copy(
                table_hbm.at[idx_v.at[pl.ds(ci * chunk, chunk)]], bufs[b], gsem
            )

        def drain(ref, sem):
            pltpu.make_async_copy(table_hbm.at[pl.ds(0, chunk)], ref, sem).wait()

        start_gather(0, 0)
        start_gather(1, 1)

        def body(g, _):
            for b in range(4):
                ci = g * 4 + b
                drain(bufs[b], gsem)
                pltpu.async_copy(
                    bufs[b], out_hbm.at[pl.ds(base + ci * chunk, chunk)], wsem
                )

                @pl.when(ci >= 2)
                def _():
                    drain(bufs[(b + 2) % 4], wsem)

                @pl.when(ci + 2 < n_chunks)
                def _():
                    start_gather(ci + 2, (b + 2) % 4)
            return 0

        lax.fori_loop(0, n_chunks // 4, body, 0)
        drain(bufs[(n_chunks - 2) % 4], wsem)
        drain(bufs[(n_chunks - 1) % 4], wsem)

    return lookup


def kernel(tokens, attention_mask, table):
    b, s = tokens.shape
    d = table.shape[1]
    idx = tokens.reshape(b * s).astype(jnp.int32)
    out = _make_lookup(b * s, d)(table, idx)
    return out.reshape(b, s, d)
